# Initial kernel scaffold; baseline (speedup 1.0000x reference)
import jax
import jax.numpy as jnp
from jax.experimental import pallas as pl


def kernel(src_feat, tgt_feat, src_xyz, tgt_xyz, W):
    raise NotImplementedError("write your pallas kernel here")


# speedup vs baseline: 1.6622x; 1.6622x over previous
import functools

import jax
import jax.numpy as jnp
from jax.experimental import pallas as pl

_B, _NS, _NT, _D = 4, 2048, 2048, 64
_RP2 = 0.25
_RN = 1.0
_BLK = 256
_NEG = -1e30


def _fused_kernel(sf_ref, tf_ref, sxyz_ref, tzt_ref, w_ref, sum_ref, cnt_ref):
    nb = pl.program_id(1)

    sf = sf_ref[0]
    tf = tf_ref[0]
    xyz = sxyz_ref[0]
    tzt = tzt_ref[0]
    w = w_ref[...]

    r = jax.lax.broadcasted_iota(jnp.int32, (_D, _D), 0)
    c = jax.lax.broadcasted_iota(jnp.int32, (_D, _D), 1)
    wt = jnp.where(r <= c, w, 0.0)
    ws = wt + wt.T

    sfw = jnp.dot(sf, ws, preferred_element_type=jnp.float32)
    logits = jax.lax.dot_general(
        sfw, tf, (((1,), (1,)), ((), ())),
        preferred_element_type=jnp.float32)

    a2 = jnp.sum(xyz * xyz, axis=1, keepdims=True)
    b2 = jnp.sum(tzt * tzt, axis=0, keepdims=True)
    ab = (xyz[:, 0:1] * tzt[0:1, :]
          + xyz[:, 1:2] * tzt[1:2, :]
          + xyz[:, 2:3] * tzt[2:3, :])
    d2 = jnp.maximum(a2 + b2 - 2.0 * ab, 0.0)

    d2min = jnp.min(d2, axis=1, keepdims=True)
    jidx = jax.lax.broadcasted_iota(jnp.int32, d2.shape, 1)
    idx1 = jnp.min(jnp.where(d2 == d2min, jidx, _NT),
                   axis=1, keepdims=True)
    onehot = jidx == idx1
    ignore = (d2 < _RN * _RN) & (~onehot)

    logits_m = jnp.where(ignore, _NEG, logits)
    pos = jnp.sum(jnp.where(onehot, logits, 0.0), axis=1, keepdims=True)
    m = jnp.max(logits_m, axis=1, keepdims=True)
    lse = m + jnp.log(jnp.sum(jnp.exp(logits_m - m), axis=1, keepdims=True))
    loss_per = lse - pos

    valid = d2min < _RP2
    psum = jnp.sum(jnp.where(valid, loss_per, 0.0))
    pcnt = jnp.sum(valid.astype(jnp.float32))

    @pl.when(nb == 0)
    def _():
        sum_ref[...] = jnp.zeros_like(sum_ref)
        cnt_ref[...] = jnp.zeros_like(cnt_ref)

    sum_ref[...] += psum
    cnt_ref[...] += pcnt


@functools.partial(jax.jit, static_argnames=("interpret",))
def kernel(src_feat, tgt_feat, src_xyz, tgt_xyz, W, interpret=False):
    nblk = _NS // _BLK
    tgt_xyz_t = jnp.swapaxes(tgt_xyz, 1, 2)

    out_shape = [
        jax.ShapeDtypeStruct((_B, 1, 128), jnp.float32),
        jax.ShapeDtypeStruct((_B, 1, 128), jnp.float32),
    ]
    grid = (_B, nblk)
    sums, cnts = pl.pallas_call(
        _fused_kernel,
        grid=grid,
        in_specs=[
            pl.BlockSpec((1, _BLK, _D), lambda b, nb: (b, nb, 0)),
            pl.BlockSpec((1, _NT, _D), lambda b, nb: (b, 0, 0)),
            pl.BlockSpec((1, _BLK, 3), lambda b, nb: (b, nb, 0)),
            pl.BlockSpec((1, 3, _NT), lambda b, nb: (b, 0, 0)),
            pl.BlockSpec((_D, _D), lambda b, nb: (0, 0)),
        ],
        out_specs=[
            pl.BlockSpec((1, 1, 128), lambda b, nb: (b, 0, 0)),
            pl.BlockSpec((1, 1, 128), lambda b, nb: (b, 0, 0)),
        ],
        out_shape=out_shape,
        interpret=interpret,
    )(src_feat, tgt_feat, src_xyz, tgt_xyz_t, W)

    loss_b = sums[:, 0, 0] / cnts[:, 0, 0]
    return jnp.mean(loss_b)
